# Initial kernel scaffold; baseline (speedup 1.0000x reference)
import jax
import jax.numpy as jnp
from jax.experimental import pallas as pl


def kernel(features, modality_ids, mode_ids, W0, b0, W1, b1, W2, b2, mode_table, Ws, bs, Wh, bh):
    raise NotImplementedError("write your pallas kernel here")


# speedup vs baseline: 1.5326x; 1.5326x over previous
import jax
import jax.numpy as jnp
from jax import lax
from jax.experimental import pallas as pl


def _tables_kernel(mt_ref, ws_ref, wh_ref, bs_ref, bh_ref, scale_ref, shift_ref):
    mt = mt_ref[...]
    dn = (((1,), (1,)), ((), ()))
    scale_ref[...] = lax.dot_general(
        mt, ws_ref[...], dn, preferred_element_type=jnp.float32) + bs_ref[...]
    shift_ref[...] = lax.dot_general(
        mt, wh_ref[...], dn, preferred_element_type=jnp.float32) + bh_ref[...]


def _main_kernel(x_ref, mod_ref, mode_ref, w0_ref, w1_ref, w2_ref,
                 b0_ref, b1_ref, b2_ref, scale_t_ref, shift_t_ref, out_ref):
    x = x_ref[...]
    d1 = w1_ref.shape[1]
    d2 = w2_ref.shape[1]
    dn = (((1,), (1,)), ((), ()))
    p0 = lax.dot_general(x, w0_ref[...], dn, preferred_element_type=jnp.float32)
    p1 = lax.dot_general(x[:, :d1], w1_ref[...], dn,
                         preferred_element_type=jnp.float32)
    p2 = lax.dot_general(x[:, :d2], w2_ref[...], dn,
                         preferred_element_type=jnp.float32)

    mids = mod_ref[0, 0, :]
    m0 = (mids == 0).astype(jnp.float32)[:, None]
    m1 = (mids == 1).astype(jnp.float32)[:, None]
    m2 = (mids == 2).astype(jnp.float32)[:, None]
    content = (m0 * p0 + m1 * p1 + m2 * p2
               + m0 * b0_ref[...] + m1 * b1_ref[...] + m2 * b2_ref[...])

    modes = mode_ref[0, 0, :]
    n_modes = scale_t_ref.shape[0]
    oh = (modes[:, None] == lax.broadcasted_iota(
        jnp.int32, (modes.shape[0], n_modes), 1)).astype(jnp.float32)
    scale = jnp.dot(oh, scale_t_ref[...], preferred_element_type=jnp.float32)
    shift = jnp.dot(oh, shift_t_ref[...], preferred_element_type=jnp.float32)

    out_ref[...] = content * (1.0 + scale) + shift


def kernel(features, modality_ids, mode_ids, W0, b0, W1, b1, W2, b2,
           mode_table, Ws, bs, Wh, bh):
    B, D = features.shape
    L = W0.shape[0]
    n_modes, mode_dim = mode_table.shape

    scale_t, shift_t = pl.pallas_call(
        _tables_kernel,
        out_shape=(jax.ShapeDtypeStruct((n_modes, L), jnp.float32),
                   jax.ShapeDtypeStruct((n_modes, L), jnp.float32)),
    )(mode_table, Ws, Wh, bs.reshape(1, L), bh.reshape(1, L))

    BM = 512
    NM = B // BM
    xb = features.astype(jnp.bfloat16)
    w0b = W0.astype(jnp.bfloat16)
    w1b = W1.astype(jnp.bfloat16)
    w2b = W2.astype(jnp.bfloat16)
    mod3 = modality_ids.reshape(NM, 1, BM)
    mode3 = mode_ids.reshape(NM, 1, BM)

    out = pl.pallas_call(
        _main_kernel,
        grid=(NM,),
        in_specs=[
            pl.BlockSpec((BM, D), lambda i: (i, 0)),
            pl.BlockSpec((1, 1, BM), lambda i: (i, 0, 0)),
            pl.BlockSpec((1, 1, BM), lambda i: (i, 0, 0)),
            pl.BlockSpec((L, D), lambda i: (0, 0)),
            pl.BlockSpec((L, W1.shape[1]), lambda i: (0, 0)),
            pl.BlockSpec((L, W2.shape[1]), lambda i: (0, 0)),
            pl.BlockSpec((1, L), lambda i: (0, 0)),
            pl.BlockSpec((1, L), lambda i: (0, 0)),
            pl.BlockSpec((1, L), lambda i: (0, 0)),
            pl.BlockSpec((n_modes, L), lambda i: (0, 0)),
            pl.BlockSpec((n_modes, L), lambda i: (0, 0)),
        ],
        out_specs=pl.BlockSpec((BM, L), lambda i: (i, 0)),
        out_shape=jax.ShapeDtypeStruct((B, L), jnp.float32),
    )(xb, mod3, mode3, w0b, w1b, w2b,
      b0.reshape(1, L), b1.reshape(1, L), b2.reshape(1, L), scale_t, shift_t)
    return out
